# baseline (speedup 1.0000x reference)
import functools

import numpy as np
import jax
import jax.numpy as jnp
from jax.experimental import pallas as pl
from jax.experimental.pallas import tpu as pltpu

_ITERS = 8
_CH = 3


def _col_masks(H, W):
    xx = np.tile(np.arange(W), H)
    m = np.zeros((4, 1, H * W), np.float32)
    m[0, 0] = (xx > 0).astype(np.float32)
    m[1, 0] = (xx < W - 1).astype(np.float32)
    m[2, 0] = 1.0
    m[3, 0] = 1.0
    return jnp.asarray(m, jnp.bfloat16)


def _wd(w):
    cout, cin = w.shape[0], w.shape[1]
    return jnp.transpose(w, (2, 0, 3, 1)).reshape(3 * cout, 3 * cin).astype(jnp.bfloat16)


def _dce_kernel(x_ref, masks_ref,
                wd1, wd2, wd3, wd4, wd5, wd6, wd7,
                xe_ref, xr_ref, *, H, W, iters):
    HW = H * W
    ml = masks_ref[0]
    mr = masks_ref[1]
    ones2 = masks_ref[2:4, 0]

    def packh(xb):
        xl = ml * pltpu.roll(xb, 1, 1)
        xr_ = mr * pltpu.roll(xb, HW - 1, 1)
        return jnp.concatenate([xl, xb, xr_], axis=0)

    def conv(packed, wd_ref, act):
        q = jnp.dot(wd_ref[...], jnp.concatenate([packed, ones2], axis=0),
                    preferred_element_type=jnp.float32)
        cout = q.shape[0] // 3
        qm, q0, qp = q[:cout], q[cout:2 * cout], q[2 * cout:]
        z = jnp.zeros((cout, W), jnp.float32)
        y = (q0 + jnp.concatenate([z, qm[:, :HW - W]], axis=1)
                + jnp.concatenate([qp[:, W:], z], axis=1))
        if act == "relu":
            return jnp.maximum(y.astype(jnp.bfloat16), jnp.bfloat16(0.0))
        return jnp.tanh(y)

    x0 = x_ref[...]
    x1 = conv(packh(x0.astype(jnp.bfloat16)), wd1, "relu")
    x2 = conv(packh(x1), wd2, "relu")
    p2 = packh(x2)
    x3 = conv(p2, wd3, "relu")
    p3 = packh(x3)
    x4 = conv(p3, wd4, "relu")
    x5 = conv(jnp.concatenate([p3, packh(x4)], axis=0), wd5, "relu")
    x6 = conv(jnp.concatenate([p2, packh(x5)], axis=0), wd6, "relu")
    xr = conv(packh(x6), wd7, "tanh")

    xe = x0
    for i in range(iters):
        ri = xr[i * _CH:(i + 1) * _CH]
        xe = jnp.clip(xe + ri * (xe * xe - xe), 0.0, 1.0)

    xe_ref[...] = xe
    xr_ref[...] = xr


def _const_spec(arr):
    zeros = (0,) * arr.ndim

    def index_map(n):
        return zeros

    return pl.BlockSpec(arr.shape, index_map)


def kernel(x, w1, b1, w2, b2, w3, b3, w4, b4, w5, b5, w6, b6, w7, b7):
    N, C, H, W = x.shape
    HW = H * W
    CR = _CH * _ITERS

    xf = x.reshape(N, C, HW).astype(jnp.float32)
    masks = _col_masks(H, W)

    def bcol(b):
        cout = b.shape[0]
        z = jnp.zeros((cout, 1), jnp.float32)
        bc = b.reshape(-1, 1)
        hi = bc.astype(jnp.bfloat16)
        lo = (bc - hi.astype(jnp.float32)).astype(jnp.bfloat16)
        zb = jnp.zeros((cout, 1), jnp.bfloat16)
        hi3 = jnp.concatenate([zb, hi, zb], axis=0)
        lo3 = jnp.concatenate([zb, lo, zb], axis=0)
        return jnp.concatenate([hi3, lo3], axis=1)

    h5 = w5.shape[1] // 2
    h6 = w6.shape[1] // 2
    wd5 = jnp.concatenate([_wd(w5[:, :h5]), _wd(w5[:, h5:]), bcol(b5)], axis=1)
    wd6 = jnp.concatenate([_wd(w6[:, :h6]), _wd(w6[:, h6:]), bcol(b6)], axis=1)

    def wb(w, b):
        return jnp.concatenate([_wd(w), bcol(b)], axis=1)

    flat = [wb(w1, b1), wb(w2, b2), wb(w3, b3), wb(w4, b4),
            wd5, wd6, wb(w7, b7)]

    body = functools.partial(_dce_kernel, H=H, W=W, iters=_ITERS)

    in_specs = ([pl.BlockSpec((None, C, HW), lambda n: (n, 0, 0)),
                 _const_spec(masks)]
                + [_const_spec(p) for p in flat])

    xe, xr = pl.pallas_call(
        body,
        out_shape=(jax.ShapeDtypeStruct((N, C, HW), jnp.float32),
                   jax.ShapeDtypeStruct((N, CR, HW), jnp.float32)),
        grid_spec=pltpu.PrefetchScalarGridSpec(
            num_scalar_prefetch=0,
            grid=(N,),
            in_specs=in_specs,
            out_specs=(pl.BlockSpec((None, C, HW), lambda n: (n, 0, 0)),
                       pl.BlockSpec((None, CR, HW), lambda n: (n, 0, 0))),
        ),
        compiler_params=pltpu.CompilerParams(
            dimension_semantics=("parallel",)),
    )(xf, masks, *flat)

    x_enhanced = xe.reshape(N, C, H, W)
    x_r = xr.reshape(N, CR, H, W)
    r = tuple(x_r[:, i * _CH:(i + 1) * _CH] for i in range(_ITERS))
    return x_enhanced, r


# speedup vs baseline: 1.0136x; 1.0136x over previous
import functools

import numpy as np
import jax
import jax.numpy as jnp
from jax.experimental import pallas as pl
from jax.experimental.pallas import tpu as pltpu

_ITERS = 8
_CH = 3


def _col_masks(H, W):
    xx = np.tile(np.arange(W), H)
    m = np.zeros((4, 1, H * W), np.float32)
    m[0, 0] = (xx > 0).astype(np.float32)
    m[1, 0] = (xx < W - 1).astype(np.float32)
    m[2, 0] = 1.0
    m[3, 0] = 1.0
    return jnp.asarray(m, jnp.bfloat16)


def _wd(w):
    cout, cin = w.shape[0], w.shape[1]
    return jnp.transpose(w, (2, 0, 3, 1)).reshape(3 * cout, 3 * cin).astype(jnp.bfloat16)


def _dce_kernel(x_ref, masks_ref,
                wd1, wd2, wd3, wd4, wd5, wd6, wd7,
                xe_ref, xr_ref, *, H, W, iters):
    HW = H * W
    ml = masks_ref[0]
    mr = masks_ref[1]
    ones2 = masks_ref[2:4, 0]

    def packh(xb):
        xl = ml * pltpu.roll(xb, 1, 1)
        xr_ = mr * pltpu.roll(xb, HW - 1, 1)
        return jnp.concatenate([xl, xb, xr_], axis=0)

    def conv(packed, wd_ref, act):
        q = jnp.dot(wd_ref[...], jnp.concatenate([packed, ones2], axis=0),
                    preferred_element_type=jnp.float32)
        cout = q.shape[0] // 3
        qm, q0, qp = q[:cout], q[cout:2 * cout], q[2 * cout:]
        z = jnp.zeros((cout, W), jnp.float32)
        y = (q0 + jnp.concatenate([z, qm[:, :HW - W]], axis=1)
                + jnp.concatenate([qp[:, W:], z], axis=1))
        if act == "relu":
            return jnp.maximum(y, 0.0).astype(jnp.bfloat16)
        return jnp.tanh(y)

    x0 = x_ref[...]
    x1 = conv(packh(x0.astype(jnp.bfloat16)), wd1, "relu")
    x2 = conv(packh(x1), wd2, "relu")
    p2 = packh(x2)
    x3 = conv(p2, wd3, "relu")
    p3 = packh(x3)
    x4 = conv(p3, wd4, "relu")
    x5 = conv(jnp.concatenate([p3, packh(x4)], axis=0), wd5, "relu")
    x6 = conv(jnp.concatenate([p2, packh(x5)], axis=0), wd6, "relu")
    xr = conv(packh(x6), wd7, "tanh")

    xe = x0
    for i in range(iters):
        ri = xr[i * _CH:(i + 1) * _CH]
        xe = jnp.clip(xe + ri * (xe * xe - xe), 0.0, 1.0)

    xe_ref[...] = xe
    xr_ref[...] = xr


def _const_spec(arr):
    zeros = (0,) * arr.ndim

    def index_map(n):
        return zeros

    return pl.BlockSpec(arr.shape, index_map)


def kernel(x, w1, b1, w2, b2, w3, b3, w4, b4, w5, b5, w6, b6, w7, b7):
    N, C, H, W = x.shape
    HW = H * W
    CR = _CH * _ITERS

    xf = x.reshape(N, C, HW).astype(jnp.float32)
    masks = _col_masks(H, W)

    def bcol(b):
        cout = b.shape[0]
        z = jnp.zeros((cout, 1), jnp.float32)
        bc = b.reshape(-1, 1)
        hi = bc.astype(jnp.bfloat16)
        lo = (bc - hi.astype(jnp.float32)).astype(jnp.bfloat16)
        zb = jnp.zeros((cout, 1), jnp.bfloat16)
        hi3 = jnp.concatenate([zb, hi, zb], axis=0)
        lo3 = jnp.concatenate([zb, lo, zb], axis=0)
        return jnp.concatenate([hi3, lo3], axis=1)

    h5 = w5.shape[1] // 2
    h6 = w6.shape[1] // 2
    wd5 = jnp.concatenate([_wd(w5[:, :h5]), _wd(w5[:, h5:]), bcol(b5)], axis=1)
    wd6 = jnp.concatenate([_wd(w6[:, :h6]), _wd(w6[:, h6:]), bcol(b6)], axis=1)

    def wb(w, b):
        return jnp.concatenate([_wd(w), bcol(b)], axis=1)

    flat = [wb(w1, b1), wb(w2, b2), wb(w3, b3), wb(w4, b4),
            wd5, wd6, wb(w7, b7)]

    body = functools.partial(_dce_kernel, H=H, W=W, iters=_ITERS)

    in_specs = ([pl.BlockSpec((None, C, HW), lambda n: (n, 0, 0)),
                 _const_spec(masks)]
                + [_const_spec(p) for p in flat])

    xe, xr = pl.pallas_call(
        body,
        out_shape=(jax.ShapeDtypeStruct((N, C, HW), jnp.float32),
                   jax.ShapeDtypeStruct((N, CR, HW), jnp.float32)),
        grid_spec=pltpu.PrefetchScalarGridSpec(
            num_scalar_prefetch=0,
            grid=(N,),
            in_specs=in_specs,
            out_specs=(pl.BlockSpec((None, C, HW), lambda n: (n, 0, 0)),
                       pl.BlockSpec((None, CR, HW), lambda n: (n, 0, 0))),
        ),
        compiler_params=pltpu.CompilerParams(
            dimension_semantics=("parallel",)),
    )(xf, masks, *flat)

    x_enhanced = xe.reshape(N, C, H, W)
    x_r = xr.reshape(N, CR, H, W)
    r = tuple(x_r[:, i * _CH:(i + 1) * _CH] for i in range(_ITERS))
    return x_enhanced, r
